# baseline (device time: 444190 ns/iter reference)
import jax
import jax.numpy as jnp
from jax import lax
from jax.experimental import pallas as pl
from jax.experimental.pallas import tpu as pltpu

N_DEV = 16


def kernel(x, w_mat):
    m, k_per = x.shape
    _, n = w_mat.shape
    m_per = m // N_DEV
    n_half = n // 2

    def body(x_ref, w_ref, out_ref, comm_a, comm_b,
             send_a, recv_a, send_b, recv_b, credit_a, credit_b):
        my = lax.axis_index("i")
        left = lax.rem(my + N_DEV - 1, N_DEV)
        right = lax.rem(my + 1, N_DEV)

        barrier_sem = pltpu.get_barrier_semaphore()
        for nbr in (left, right):
            pl.semaphore_signal(barrier_sem, inc=1, device_id=(nbr,),
                                device_id_type=pl.DeviceIdType.MESH)
        pl.semaphore_wait(barrier_sem, 2)

        def partial_a(c):
            xa = x_ref[pl.ds(c * m_per, m_per), :]
            return jnp.dot(xa, w_ref[:, 0:n_half],
                           preferred_element_type=jnp.float32).astype(jnp.bfloat16)

        def partial_b(c):
            xa = x_ref[pl.ds(c * m_per, m_per), :]
            return jnp.dot(xa, w_ref[:, n_half:n],
                           preferred_element_type=jnp.float32).astype(jnp.bfloat16)

        comm_a[0, :, :] = partial_a(lax.rem(my + N_DEV - 1, N_DEV))
        comm_b[0, :, :] = partial_b(lax.rem(my + 1, N_DEV))

        acc_a = None
        acc_b = None
        for h in range(N_DEV - 1):
            s = h % 2
            r = (h + 1) % 2
            if h >= 2:
                pl.semaphore_wait(credit_a, 1)
                pl.semaphore_wait(credit_b, 1)
            rdma_a = pltpu.make_async_remote_copy(
                src_ref=comm_a.at[s],
                dst_ref=comm_a.at[r],
                send_sem=send_a.at[s],
                recv_sem=recv_a.at[r],
                device_id=(right,),
                device_id_type=pl.DeviceIdType.MESH,
            )
            rdma_b = pltpu.make_async_remote_copy(
                src_ref=comm_b.at[s],
                dst_ref=comm_b.at[r],
                send_sem=send_b.at[s],
                recv_sem=recv_b.at[r],
                device_id=(left,),
                device_id_type=pl.DeviceIdType.MESH,
            )
            rdma_a.start()
            rdma_b.start()
            part_a = partial_a(lax.rem(my + 2 * N_DEV - 2 - h, N_DEV))
            part_b = partial_b(lax.rem(my + 2 + h, N_DEV))
            rdma_a.wait()
            rdma_b.wait()
            acc_a = comm_a[r, :, :].astype(jnp.float32) + part_a.astype(jnp.float32)
            acc_b = comm_b[r, :, :].astype(jnp.float32) + part_b.astype(jnp.float32)
            if h < N_DEV - 2:
                comm_a[r, :, :] = acc_a.astype(jnp.bfloat16)
                comm_b[r, :, :] = acc_b.astype(jnp.bfloat16)
            if 1 <= h <= N_DEV - 3:
                pl.semaphore_signal(credit_a, inc=1, device_id=(left,),
                                    device_id_type=pl.DeviceIdType.MESH)
                pl.semaphore_signal(credit_b, inc=1, device_id=(right,),
                                    device_id_type=pl.DeviceIdType.MESH)

        c = 0.7978845608028654
        ya, yb = acc_a, acc_b
        out_ref[:, 0:n_half] = 0.5 * ya * (1.0 + jnp.tanh(c * (ya + 0.044715 * ya * ya * ya)))
        out_ref[:, n_half:n] = 0.5 * yb * (1.0 + jnp.tanh(c * (yb + 0.044715 * yb * yb * yb)))

    return pl.pallas_call(
        body,
        out_shape=jax.ShapeDtypeStruct((m_per, n), jnp.float32),
        in_specs=[
            pl.BlockSpec(memory_space=pltpu.VMEM),
            pl.BlockSpec(memory_space=pltpu.VMEM),
        ],
        out_specs=pl.BlockSpec(memory_space=pltpu.VMEM),
        scratch_shapes=[
            pltpu.VMEM((2, m_per, n_half), jnp.bfloat16),
            pltpu.VMEM((2, m_per, n_half), jnp.bfloat16),
            pltpu.SemaphoreType.DMA((2,)),
            pltpu.SemaphoreType.DMA((2,)),
            pltpu.SemaphoreType.DMA((2,)),
            pltpu.SemaphoreType.DMA((2,)),
            pltpu.SemaphoreType.REGULAR,
            pltpu.SemaphoreType.REGULAR,
        ],
        compiler_params=pltpu.CompilerParams(collective_id=0),
    )(x, w_mat)


# device time: 358909 ns/iter; 1.2376x vs baseline; 1.2376x over previous
import jax
import jax.numpy as jnp
from jax import lax
from jax.experimental import pallas as pl
from jax.experimental.pallas import tpu as pltpu

N_DEV = 16
N_SUB = 2


def kernel(x, w_mat):
    m, k_per = x.shape
    _, n = w_mat.shape
    m_per = m // N_DEV
    n_half = n // 2
    n_sub = n_half // N_SUB

    def body(x_ref, w_ref, out_ref,
             comm_a0, comm_a1, comm_b0, comm_b1,
             send_a0, recv_a0, send_a1, recv_a1,
             send_b0, recv_b0, send_b1, recv_b1,
             credit_a0, credit_a1, credit_b0, credit_b1):
        my = lax.axis_index("i")
        left = lax.rem(my + N_DEV - 1, N_DEV)
        right = lax.rem(my + 1, N_DEV)

        rings = [
            (comm_a0, send_a0, recv_a0, credit_a0, right, left, 0),
            (comm_a1, send_a1, recv_a1, credit_a1, right, left, n_sub),
            (comm_b0, send_b0, recv_b0, credit_b0, left, right, n_half),
            (comm_b1, send_b1, recv_b1, credit_b1, left, right, n_half + n_sub),
        ]

        barrier_sem = pltpu.get_barrier_semaphore()
        for nbr in (left, right):
            pl.semaphore_signal(barrier_sem, inc=1, device_id=(nbr,),
                                device_id_type=pl.DeviceIdType.MESH)
        pl.semaphore_wait(barrier_sem, 2)

        def partial_a(c):
            xa = x_ref[pl.ds(c * m_per, m_per), :]
            return jnp.dot(xa, w_ref[:, 0:n_half],
                           preferred_element_type=jnp.float32)

        def partial_b(c):
            xa = x_ref[pl.ds(c * m_per, m_per), :]
            return jnp.dot(xa, w_ref[:, n_half:n],
                           preferred_element_type=jnp.float32)

        pa = partial_a(lax.rem(my + N_DEV - 1, N_DEV)).astype(jnp.bfloat16)
        pb = partial_b(lax.rem(my + 1, N_DEV)).astype(jnp.bfloat16)
        comm_a0[0, :, :] = pa[:, 0:n_sub]
        comm_a1[0, :, :] = pa[:, n_sub:n_half]
        comm_b0[0, :, :] = pb[:, 0:n_sub]
        comm_b1[0, :, :] = pb[:, n_sub:n_half]

        inflight = [[] for _ in rings]
        for i, (comm, ssem, rsem, _, down, _, _) in enumerate(rings):
            rdma = pltpu.make_async_remote_copy(
                src_ref=comm.at[0], dst_ref=comm.at[1],
                send_sem=ssem.at[0], recv_sem=rsem.at[1],
                device_id=(down,), device_id_type=pl.DeviceIdType.MESH,
            )
            rdma.start()
            inflight[i].append(rdma)

        for h in range(N_DEV - 1):
            r = (h + 1) % 3
            d2 = (h + 2) % 3
            part_a = partial_a(lax.rem(my + 2 * N_DEV - 2 - h, N_DEV))
            part_b = partial_b(lax.rem(my + 2 + h, N_DEV))
            parts = (part_a[:, 0:n_sub], part_a[:, n_sub:n_half],
                     part_b[:, 0:n_sub], part_b[:, n_sub:n_half])
            for i, (comm, ssem, rsem, credit, down, up, col) in enumerate(rings):
                if h >= 1:
                    inflight[i].pop(0).wait_send()
                    if h <= N_DEV - 3:
                        pl.semaphore_signal(
                            credit, inc=1, device_id=(up,),
                            device_id_type=pl.DeviceIdType.MESH)
                recv = pltpu.make_async_remote_copy(
                    src_ref=comm.at[d2], dst_ref=comm.at[r],
                    send_sem=ssem.at[d2], recv_sem=rsem.at[r],
                    device_id=(up,), device_id_type=pl.DeviceIdType.MESH,
                )
                recv.wait_recv()
                acc = comm[r, :, :].astype(jnp.float32) + parts[i].astype(jnp.float32)
                if h < N_DEV - 2:
                    comm[r, :, :] = acc.astype(jnp.bfloat16)
                    if h >= 1:
                        pl.semaphore_wait(credit, 1)
                    rdma = pltpu.make_async_remote_copy(
                        src_ref=comm.at[r], dst_ref=comm.at[d2],
                        send_sem=ssem.at[r], recv_sem=rsem.at[d2],
                        device_id=(down,), device_id_type=pl.DeviceIdType.MESH,
                    )
                    rdma.start()
                    inflight[i].append(rdma)
                else:
                    y = acc
                    c = 0.7978845608028654
                    out_ref[:, pl.ds(col, n_sub)] = (
                        0.5 * y * (1.0 + jnp.tanh(c * (y + 0.044715 * y * y * y))))

        for q in inflight:
            for rdma in q:
                rdma.wait_send()

    return pl.pallas_call(
        body,
        out_shape=jax.ShapeDtypeStruct((m_per, n), jnp.float32),
        in_specs=[
            pl.BlockSpec(memory_space=pltpu.VMEM),
            pl.BlockSpec(memory_space=pltpu.VMEM),
        ],
        out_specs=pl.BlockSpec(memory_space=pltpu.VMEM),
        scratch_shapes=[
            pltpu.VMEM((3, m_per, n_sub), jnp.bfloat16),
            pltpu.VMEM((3, m_per, n_sub), jnp.bfloat16),
            pltpu.VMEM((3, m_per, n_sub), jnp.bfloat16),
            pltpu.VMEM((3, m_per, n_sub), jnp.bfloat16),
            pltpu.SemaphoreType.DMA((3,)), pltpu.SemaphoreType.DMA((3,)),
            pltpu.SemaphoreType.DMA((3,)), pltpu.SemaphoreType.DMA((3,)),
            pltpu.SemaphoreType.DMA((3,)), pltpu.SemaphoreType.DMA((3,)),
            pltpu.SemaphoreType.DMA((3,)), pltpu.SemaphoreType.DMA((3,)),
            pltpu.SemaphoreType.REGULAR, pltpu.SemaphoreType.REGULAR,
            pltpu.SemaphoreType.REGULAR, pltpu.SemaphoreType.REGULAR,
        ],
        compiler_params=pltpu.CompilerParams(collective_id=0),
    )(x, w_mat)
